# baseline (speedup 1.0000x reference)
import functools

import jax
import jax.numpy as jnp
from jax import lax
from jax.experimental import pallas as pl
from jax.experimental.pallas import tpu as pltpu
from jax.experimental.pallas import tpu_sc as plsc

NC = 2
NS = 16
NW = NC * NS
LANES = 16

B = 16384
L = 200
CW = B // NW
RT = 8
N_UNITS = L // RT
GPR = CW // LANES


def _body(dense_hbm, cat_hbm, emb_hbm, wb_hbm, out_hbm,
          emb_v, wb_v, lut_v, d0, d1, c0, c1, o0, o1,
          sd0, sd1, sc0, sc1, so0, so1):
    wid = lax.axis_index("s") * NC + lax.axis_index("c")
    b0 = wid * CW
    q0 = wid * (CW // 128)
    dbufs, cbufs, obufs = [d0, d1], [c0, c1], [o0, o1]
    dsems, csems, osems = [sd0, sd1], [sc0, sc1], [so0, so1]

    pltpu.sync_copy(emb_hbm, emb_v)
    pltpu.sync_copy(wb_hbm, wb_v)
    iota = lax.iota(jnp.int32, LANES)
    e0 = plsc.load_gather(emb_v, [jnp.minimum(iota * 2, 14)])
    e1 = plsc.load_gather(emb_v, [jnp.minimum(iota * 2 + 1, 15)])
    nw0v = wb_v[pl.ds(0, LANES)]
    w1v = wb_v[pl.ds(LANES, LANES)]
    w2v = wb_v[pl.ds(2 * LANES, LANES)]
    bv = wb_v[pl.ds(3 * LANES, LANES)]
    lut_v[...] = -(e0 * w1v + e1 * w2v + bv)

    def start_in(u, k):
        r = u * RT
        pltpu.async_copy(
            dense_hbm.at[pl.ds(r, RT), pl.ds(b0, CW)], dbufs[k], dsems[k])
        pltpu.async_copy(
            cat_hbm.at[pl.ds(r, RT), pl.ds(b0, CW)], cbufs[k], csems[k])

    def wait_in(k):
        pltpu.make_async_copy(
            dense_hbm.at[pl.ds(0, RT), pl.ds(b0, CW)], dbufs[k], dsems[k]).wait()
        pltpu.make_async_copy(
            cat_hbm.at[pl.ds(0, RT), pl.ds(b0, CW)], cbufs[k], csems[k]).wait()

    def start_out(u, k):
        pltpu.async_copy(
            obufs[k],
            out_hbm.at[pl.ds(u * RT, RT), pl.ds(q0, CW // 128)],
            osems[k])

    def wait_out(k):
        pltpu.make_async_copy(
            obufs[k],
            out_hbm.at[pl.ds(0, RT), pl.ds(q0, CW // 128)],
            osems[k]).wait()

    def compute(k):
        dbuf, cbuf, obuf = dbufs[k], cbufs[k], obufs[k]

        @plsc.parallel_loop(0, RT, step=1, unroll=2)
        def _(i):
            for g in range(GPR):
                s = pl.ds(g * LANES, LANES)
                d = dbuf[i, s]
                c = cbuf[i, s]
                nt = plsc.load_gather(lut_v, [c])
                y = 1.0 / (1.0 + jnp.exp(d * nw0v + nt))
                obuf[i, g // 8, pl.ds((g % 8) * LANES, LANES)] = y

    n_pairs = (N_UNITS - 1) // 2
    start_in(0, 0)

    def pair_body(p, _):
        u0 = 2 * p
        start_in(u0 + 1, 1)
        wait_in(0)

        @pl.when(p > 0)
        def _():
            wait_out(0)

        compute(0)
        start_out(u0, 0)
        start_in(u0 + 2, 0)
        wait_in(1)

        @pl.when(p > 0)
        def _():
            wait_out(1)

        compute(1)
        start_out(u0 + 1, 1)
        return 0

    lax.fori_loop(0, n_pairs, pair_body, 0)
    wait_in(0)
    wait_out(0)
    compute(0)
    start_out(N_UNITS - 1, 0)
    wait_out(1)
    wait_out(0)


_mesh = plsc.VectorSubcoreMesh(
    core_axis_name="c", subcore_axis_name="s", num_cores=NC, num_subcores=NS)

_sc_call = functools.partial(
    pl.kernel,
    out_type=jax.ShapeDtypeStruct((L, 128, B // 128), jnp.float32),
    mesh=_mesh,
    compiler_params=pltpu.CompilerParams(
        needs_layout_passes=False, use_tc_tiling_on_sc=True),
    scratch_types=[
        pltpu.VMEM((LANES,), jnp.float32),
        pltpu.VMEM((4 * LANES,), jnp.float32),
        pltpu.VMEM((LANES,), jnp.float32),
        pltpu.VMEM((RT, CW), jnp.float32),
        pltpu.VMEM((RT, CW), jnp.float32),
        pltpu.VMEM((RT, CW), jnp.int32),
        pltpu.VMEM((RT, CW), jnp.int32),
        pltpu.VMEM((RT, CW // 128, 128), jnp.float32),
        pltpu.VMEM((RT, CW // 128, 128), jnp.float32),
        pltpu.SemaphoreType.DMA,
        pltpu.SemaphoreType.DMA,
        pltpu.SemaphoreType.DMA,
        pltpu.SemaphoreType.DMA,
        pltpu.SemaphoreType.DMA,
        pltpu.SemaphoreType.DMA,
    ],
)(_body)


def kernel(denseFeat, catFeat, emb_table, W, b):
    dT = denseFeat.T
    cT = catFeat.astype(jnp.int32).T
    emb16 = jnp.zeros((LANES,), jnp.float32).at[:10].set(emb_table.reshape(-1))
    wb64 = jnp.concatenate([
        jnp.broadcast_to(-W[0, 0], (LANES,)),
        jnp.broadcast_to(W[1, 0], (LANES,)),
        jnp.broadcast_to(W[2, 0], (LANES,)),
        jnp.broadcast_to(b[0], (LANES,)),
    ]).astype(jnp.float32)
    out3 = _sc_call(dT, cT, emb16, wb64)
    return out3.reshape(L, B).T[..., None]


# speedup vs baseline: 1.0986x; 1.0986x over previous
import functools

import jax
import jax.numpy as jnp
from jax import lax
from jax.experimental import pallas as pl
from jax.experimental.pallas import tpu as pltpu
from jax.experimental.pallas import tpu_sc as plsc

NC = 2
NS = 16
NW = NC * NS
LANES = 16

B = 16384
L = 200
CW = B // NW
RT = 8
N_UNITS = L // RT
GPR = CW // LANES


def _body(dense_hbm, cat_hbm, emb_hbm, wb_hbm, out_hbm,
          emb_v, wb_v, lut_v, d0, d1, c0, c1, o0, o1,
          sd0, sd1, sc0, sc1, so0, so1):
    wid = lax.axis_index("s") * NC + lax.axis_index("c")
    b0 = wid * CW
    q0 = wid * (CW // 128)
    dbufs, cbufs, obufs = [d0, d1], [c0, c1], [o0, o1]
    dsems, csems, osems = [sd0, sd1], [sc0, sc1], [so0, so1]

    pltpu.sync_copy(emb_hbm, emb_v)
    pltpu.sync_copy(wb_hbm, wb_v)
    iota = lax.iota(jnp.int32, LANES)
    e0 = plsc.load_gather(emb_v, [jnp.minimum(iota * 2, 14)])
    e1 = plsc.load_gather(emb_v, [jnp.minimum(iota * 2 + 1, 15)])
    nw0v = wb_v[pl.ds(0, LANES)]
    w1v = wb_v[pl.ds(LANES, LANES)]
    w2v = wb_v[pl.ds(2 * LANES, LANES)]
    bv = wb_v[pl.ds(3 * LANES, LANES)]
    lut_v[...] = -(e0 * w1v + e1 * w2v + bv)

    def start_in(u, k):
        r = u * RT
        pltpu.async_copy(
            dense_hbm.at[pl.ds(r, RT), pl.ds(b0, CW)], dbufs[k], dsems[k])
        pltpu.async_copy(
            cat_hbm.at[pl.ds(r, RT), pl.ds(b0, CW)], cbufs[k], csems[k])

    def wait_in(k):
        pltpu.make_async_copy(
            dense_hbm.at[pl.ds(0, RT), pl.ds(b0, CW)], dbufs[k], dsems[k]).wait()
        pltpu.make_async_copy(
            cat_hbm.at[pl.ds(0, RT), pl.ds(b0, CW)], cbufs[k], csems[k]).wait()

    def start_out(u, k):
        pltpu.async_copy(
            obufs[k],
            out_hbm.at[pl.ds(u * RT, RT), pl.ds(b0, CW)],
            osems[k])

    def wait_out(k):
        pltpu.make_async_copy(
            obufs[k],
            out_hbm.at[pl.ds(0, RT), pl.ds(b0, CW)],
            osems[k]).wait()

    def compute(k):
        dbuf, cbuf, obuf = dbufs[k], cbufs[k], obufs[k]

        @plsc.parallel_loop(0, RT, step=1, unroll=2)
        def _(i):
            for g in range(GPR):
                s = pl.ds(g * LANES, LANES)
                d = dbuf[i, s]
                c = cbuf[i, s]
                nt = plsc.load_gather(lut_v, [c])
                obuf[i, s] = 1.0 / (1.0 + jnp.exp(d * nw0v + nt))

    n_pairs = (N_UNITS - 1) // 2
    start_in(0, 0)

    def pair_body(p, _):
        u0 = 2 * p
        start_in(u0 + 1, 1)
        wait_in(0)

        @pl.when(p > 0)
        def _():
            wait_out(0)

        compute(0)
        start_out(u0, 0)
        start_in(u0 + 2, 0)
        wait_in(1)

        @pl.when(p > 0)
        def _():
            wait_out(1)

        compute(1)
        start_out(u0 + 1, 1)
        return 0

    lax.fori_loop(0, n_pairs, pair_body, 0)
    wait_in(0)
    wait_out(0)
    compute(0)
    start_out(N_UNITS - 1, 0)
    wait_out(1)
    wait_out(0)


_mesh = plsc.VectorSubcoreMesh(
    core_axis_name="c", subcore_axis_name="s", num_cores=NC, num_subcores=NS)

_sc_call = functools.partial(
    pl.kernel,
    out_type=jax.ShapeDtypeStruct((L, B), jnp.float32),
    mesh=_mesh,
    compiler_params=pltpu.CompilerParams(
        needs_layout_passes=False, use_tc_tiling_on_sc=True),
    scratch_types=[
        pltpu.VMEM((LANES,), jnp.float32),
        pltpu.VMEM((4 * LANES,), jnp.float32),
        pltpu.VMEM((LANES,), jnp.float32),
        pltpu.VMEM((RT, CW), jnp.float32),
        pltpu.VMEM((RT, CW), jnp.float32),
        pltpu.VMEM((RT, CW), jnp.int32),
        pltpu.VMEM((RT, CW), jnp.int32),
        pltpu.VMEM((RT, CW), jnp.float32),
        pltpu.VMEM((RT, CW), jnp.float32),
        pltpu.SemaphoreType.DMA,
        pltpu.SemaphoreType.DMA,
        pltpu.SemaphoreType.DMA,
        pltpu.SemaphoreType.DMA,
        pltpu.SemaphoreType.DMA,
        pltpu.SemaphoreType.DMA,
    ],
)(_body)


def kernel(denseFeat, catFeat, emb_table, W, b):
    dT = denseFeat.T
    cT = catFeat.astype(jnp.int32).T
    emb16 = jnp.zeros((LANES,), jnp.float32).at[:10].set(emb_table.reshape(-1))
    wb64 = jnp.concatenate([
        jnp.broadcast_to(-W[0, 0], (LANES,)),
        jnp.broadcast_to(W[1, 0], (LANES,)),
        jnp.broadcast_to(W[2, 0], (LANES,)),
        jnp.broadcast_to(b[0], (LANES,)),
    ]).astype(jnp.float32)
    out2 = _sc_call(dT, cT, emb16, wb64)
    return out2.T[..., None]


# speedup vs baseline: 2.3871x; 2.1728x over previous
import functools

import jax
import jax.numpy as jnp
from jax import lax
from jax.experimental import pallas as pl
from jax.experimental.pallas import tpu as pltpu
from jax.experimental.pallas import tpu_sc as plsc

NC = 2
NS = 16
NW = NC * NS
LANES = 16

B = 16384
L = 200
CW = B // NW
RT = 40
N_UNITS = L // RT
NELEM = RT * CW


def _body(dense_hbm, cat_hbm, emb_hbm, wb_hbm, out_hbm,
          emb_v, wb_v, lut_v, d0, d1, c0, c1, o0, o1,
          sd0, sd1, sc0, sc1, so0, so1):
    wid = lax.axis_index("s") * NC + lax.axis_index("c")
    b0 = wid * CW
    q0 = wid * (CW // 128)
    dbufs, cbufs, obufs = [d0, d1], [c0, c1], [o0, o1]
    dsems, csems, osems = [sd0, sd1], [sc0, sc1], [so0, so1]

    pltpu.sync_copy(emb_hbm, emb_v)
    pltpu.sync_copy(wb_hbm, wb_v)
    iota = lax.iota(jnp.int32, LANES)
    e0 = plsc.load_gather(emb_v, [jnp.minimum(iota * 2, 14)])
    e1 = plsc.load_gather(emb_v, [jnp.minimum(iota * 2 + 1, 15)])
    nw0v = wb_v[pl.ds(0, LANES)]
    w1v = wb_v[pl.ds(LANES, LANES)]
    w2v = wb_v[pl.ds(2 * LANES, LANES)]
    bv = wb_v[pl.ds(3 * LANES, LANES)]
    lut_v[...] = -(e0 * w1v + e1 * w2v + bv)

    def start_in(u, k):
        r = u * RT
        pltpu.async_copy(
            dense_hbm.at[pl.ds(r, RT), pl.ds(b0, CW)], dbufs[k], dsems[k])
        pltpu.async_copy(
            cat_hbm.at[pl.ds(r, RT), pl.ds(b0, CW)], cbufs[k], csems[k])

    def wait_in(k):
        pltpu.make_async_copy(
            dense_hbm.at[pl.ds(0, RT), pl.ds(b0, CW)], dbufs[k], dsems[k]).wait()
        pltpu.make_async_copy(
            cat_hbm.at[pl.ds(0, RT), pl.ds(b0, CW)], cbufs[k], csems[k]).wait()

    def start_out(u, k):
        pltpu.async_copy(
            obufs[k],
            out_hbm.at[pl.ds(u * RT, RT), pl.ds(b0, CW)],
            osems[k])

    def wait_out(k):
        pltpu.make_async_copy(
            obufs[k],
            out_hbm.at[pl.ds(0, RT), pl.ds(b0, CW)],
            osems[k]).wait()

    def compute(k):
        dbuf, cbuf, obuf = dbufs[k], cbufs[k], obufs[k]

        @plsc.parallel_loop(0, NELEM, step=LANES, unroll=8)
        def _(i):
            r = i // CW
            s = pl.ds(i - r * CW, LANES)
            d = dbuf[r, s]
            c = cbuf[r, s]
            nt = plsc.load_gather(lut_v, [c])
            obuf[r, s] = 1.0 / (1.0 + jnp.exp(d * nw0v + nt))

    n_pairs = (N_UNITS - 1) // 2
    start_in(0, 0)

    def pair_body(p, _):
        u0 = 2 * p
        start_in(u0 + 1, 1)
        wait_in(0)

        @pl.when(p > 0)
        def _():
            wait_out(0)

        compute(0)
        start_out(u0, 0)
        start_in(u0 + 2, 0)
        wait_in(1)

        @pl.when(p > 0)
        def _():
            wait_out(1)

        compute(1)
        start_out(u0 + 1, 1)
        return 0

    lax.fori_loop(0, n_pairs, pair_body, 0)
    wait_in(0)
    wait_out(0)
    compute(0)
    start_out(N_UNITS - 1, 0)
    wait_out(1)
    wait_out(0)


_mesh = plsc.VectorSubcoreMesh(
    core_axis_name="c", subcore_axis_name="s", num_cores=NC, num_subcores=NS)

_sc_call = functools.partial(
    pl.kernel,
    out_type=jax.ShapeDtypeStruct((L, B), jnp.float32),
    mesh=_mesh,
    compiler_params=pltpu.CompilerParams(
        needs_layout_passes=False, use_tc_tiling_on_sc=True),
    scratch_types=[
        pltpu.VMEM((LANES,), jnp.float32),
        pltpu.VMEM((4 * LANES,), jnp.float32),
        pltpu.VMEM((LANES,), jnp.float32),
        pltpu.VMEM((RT, CW), jnp.float32),
        pltpu.VMEM((RT, CW), jnp.float32),
        pltpu.VMEM((RT, CW), jnp.int32),
        pltpu.VMEM((RT, CW), jnp.int32),
        pltpu.VMEM((RT, CW), jnp.float32),
        pltpu.VMEM((RT, CW), jnp.float32),
        pltpu.SemaphoreType.DMA,
        pltpu.SemaphoreType.DMA,
        pltpu.SemaphoreType.DMA,
        pltpu.SemaphoreType.DMA,
        pltpu.SemaphoreType.DMA,
        pltpu.SemaphoreType.DMA,
    ],
)(_body)


def kernel(denseFeat, catFeat, emb_table, W, b):
    dT = denseFeat.T
    cT = catFeat.astype(jnp.int32).T
    emb16 = jnp.zeros((LANES,), jnp.float32).at[:10].set(emb_table.reshape(-1))
    wb64 = jnp.concatenate([
        jnp.broadcast_to(-W[0, 0], (LANES,)),
        jnp.broadcast_to(W[1, 0], (LANES,)),
        jnp.broadcast_to(W[2, 0], (LANES,)),
        jnp.broadcast_to(b[0], (LANES,)),
    ]).astype(jnp.float32)
    out2 = _sc_call(dT, cT, emb16, wb64)
    return out2.T[..., None]


# speedup vs baseline: 2.4423x; 1.0231x over previous
import functools

import jax
import jax.numpy as jnp
from jax import lax
from jax.experimental import pallas as pl
from jax.experimental.pallas import tpu as pltpu
from jax.experimental.pallas import tpu_sc as plsc

NC = 2
NS = 16
NW = NC * NS
LANES = 16

B = 16384
L = 200
CW = B // NW
RT = 40
N_UNITS = L // RT
NELEM = RT * CW


def _body(dense_hbm, cat_hbm, emb_hbm, wb_hbm, out_hbm,
          emb_v, wb_v, lut_v, d0, d1, c0, c1, o0, o1,
          sd0, sd1, sc0, sc1, so0, so1):
    wid = lax.axis_index("s") * NC + lax.axis_index("c")
    b0 = wid * CW
    q0 = wid * (CW // 128)
    dbufs, cbufs, obufs = [d0, d1], [c0, c1], [o0, o1]
    dsems, csems, osems = [sd0, sd1], [sc0, sc1], [so0, so1]

    pltpu.sync_copy(emb_hbm, emb_v)
    pltpu.sync_copy(wb_hbm, wb_v)
    iota = lax.iota(jnp.int32, LANES)
    e0 = plsc.load_gather(emb_v, [jnp.minimum(iota * 2, 14)])
    e1 = plsc.load_gather(emb_v, [jnp.minimum(iota * 2 + 1, 15)])
    nw0v = wb_v[pl.ds(0, LANES)]
    w1v = wb_v[pl.ds(LANES, LANES)]
    w2v = wb_v[pl.ds(2 * LANES, LANES)]
    bv = wb_v[pl.ds(3 * LANES, LANES)]
    lut_v[...] = -(e0 * w1v + e1 * w2v + bv)

    def start_in(u, k):
        r = u * RT
        pltpu.async_copy(
            dense_hbm.at[pl.ds(r, RT), pl.ds(b0, CW)], dbufs[k], dsems[k])
        pltpu.async_copy(
            cat_hbm.at[pl.ds(r, RT), pl.ds(b0, CW)], cbufs[k], csems[k])

    def wait_in(k):
        pltpu.make_async_copy(
            dense_hbm.at[pl.ds(0, RT), pl.ds(b0, CW)], dbufs[k], dsems[k]).wait()
        pltpu.make_async_copy(
            cat_hbm.at[pl.ds(0, RT), pl.ds(b0, CW)], cbufs[k], csems[k]).wait()

    def start_out(u, k):
        pltpu.async_copy(
            obufs[k],
            out_hbm.at[pl.ds(u * RT, RT), pl.ds(q0, CW // 128)],
            osems[k])

    def wait_out(k):
        pltpu.make_async_copy(
            obufs[k],
            out_hbm.at[pl.ds(0, RT), pl.ds(q0, CW // 128)],
            osems[k]).wait()

    def compute(k):
        dbuf, cbuf, obuf = dbufs[k], cbufs[k], obufs[k]

        @plsc.parallel_loop(0, NELEM, step=LANES, unroll=8)
        def _(i):
            r = i // CW
            c2 = i - r * CW
            s = pl.ds(c2, LANES)
            d = dbuf[r, s]
            c = cbuf[r, s]
            nt = plsc.load_gather(lut_v, [c])
            q = c2 // 128
            obuf[r, q, pl.ds(c2 - q * 128, LANES)] = (
                1.0 / (1.0 + jnp.exp(d * nw0v + nt)))

    n_pairs = (N_UNITS - 1) // 2
    start_in(0, 0)

    def pair_body(p, _):
        u0 = 2 * p
        start_in(u0 + 1, 1)
        wait_in(0)

        @pl.when(p > 0)
        def _():
            wait_out(0)

        compute(0)
        start_out(u0, 0)
        start_in(u0 + 2, 0)
        wait_in(1)

        @pl.when(p > 0)
        def _():
            wait_out(1)

        compute(1)
        start_out(u0 + 1, 1)
        return 0

    lax.fori_loop(0, n_pairs, pair_body, 0)
    wait_in(0)
    wait_out(0)
    compute(0)
    start_out(N_UNITS - 1, 0)
    wait_out(1)
    wait_out(0)


_mesh = plsc.VectorSubcoreMesh(
    core_axis_name="c", subcore_axis_name="s", num_cores=NC, num_subcores=NS)

_sc_call = functools.partial(
    pl.kernel,
    out_type=jax.ShapeDtypeStruct((L, 128, B // 128), jnp.float32),
    mesh=_mesh,
    compiler_params=pltpu.CompilerParams(
        needs_layout_passes=False, use_tc_tiling_on_sc=True),
    scratch_types=[
        pltpu.VMEM((LANES,), jnp.float32),
        pltpu.VMEM((4 * LANES,), jnp.float32),
        pltpu.VMEM((LANES,), jnp.float32),
        pltpu.VMEM((RT, CW), jnp.float32),
        pltpu.VMEM((RT, CW), jnp.float32),
        pltpu.VMEM((RT, CW), jnp.int32),
        pltpu.VMEM((RT, CW), jnp.int32),
        pltpu.VMEM((RT, CW // 128, 128), jnp.float32),
        pltpu.VMEM((RT, CW // 128, 128), jnp.float32),
        pltpu.SemaphoreType.DMA,
        pltpu.SemaphoreType.DMA,
        pltpu.SemaphoreType.DMA,
        pltpu.SemaphoreType.DMA,
        pltpu.SemaphoreType.DMA,
        pltpu.SemaphoreType.DMA,
    ],
)(_body)


def kernel(denseFeat, catFeat, emb_table, W, b):
    dT = denseFeat.T
    cT = catFeat.astype(jnp.int32).T
    emb16 = jnp.zeros((LANES,), jnp.float32).at[:10].set(emb_table.reshape(-1))
    wb64 = jnp.concatenate([
        jnp.broadcast_to(-W[0, 0], (LANES,)),
        jnp.broadcast_to(W[1, 0], (LANES,)),
        jnp.broadcast_to(W[2, 0], (LANES,)),
        jnp.broadcast_to(b[0], (LANES,)),
    ]).astype(jnp.float32)
    out3 = _sc_call(dT, cT, emb16, wb64)
    return jax.lax.reshape(out3, (B, L, 1), dimensions=(1, 2, 0))
